# baseline (device time: 15735 ns/iter reference)
import jax
import jax.numpy as jnp
from jax import lax
from jax.experimental import pallas as pl
from jax.experimental.pallas import tpu as pltpu

N_DEV = 4
B, SQ, SKV, HQ, DH = 2, 128, 512, 4, 64
D_MODEL = 512
SKV_SH = SKV // N_DEV
GR = 32
PPB = HQ + 1
NP = B * PPB


def _batch_planes(q, k_ref, v_ref, b, rows, mask):
    ctx_list, m_list, l_list = [], [], []
    for h in range(HQ):
        bh = b * HQ + h
        qbh = q[b * SQ:b * SQ + rows, h * DH:(h + 1) * DH]
        s = lax.dot_general(
            qbh, k_ref[bh], (((1,), (1,)), ((), ())),
            preferred_element_type=jnp.float32) * 0.125
        if mask is not None:
            s = jnp.where(mask, s, -1e9)
        m = jnp.max(s, axis=1, keepdims=True)
        w = jnp.exp(s - m)
        l = jnp.sum(w, axis=1, keepdims=True)
        ctx_list.append(jnp.dot(w, v_ref[bh],
                                preferred_element_type=jnp.float32))
        m_list.append(m)
        l_list.append(l)
    stats = jnp.concatenate(
        m_list + l_list + [jnp.zeros((rows, DH - 2 * HQ), jnp.float32)],
        axis=1)
    payload = jnp.concatenate([jnp.stack(ctx_list, axis=0), stats[None]],
                              axis=0)
    return payload.astype(jnp.bfloat16)


def _body(x_ref, wq_ref, k_ref, v_ref, wo_ref, out_ref,
          buf, send_sems, recv_sems):
    my = lax.axis_index("i")
    d_order = (2, 1, 3)
    w_order = (1, 3, 2)

    barrier = pltpu.get_barrier_semaphore()
    for d in range(1, N_DEV):
        peer = lax.rem(my + d, N_DEV)
        pl.semaphore_signal(barrier, inc=1, device_id=(peer,),
                            device_id_type=pl.DeviceIdType.MESH)

    q = jnp.dot(x_ref[...], wq_ref[...],
                preferred_element_type=jnp.float32)

    qi = lax.broadcasted_iota(jnp.int32, (SQ, SKV_SH), 0)
    kj = lax.broadcasted_iota(jnp.int32, (SQ, SKV_SH), 1)
    mask1 = (kj <= qi) | (qi < GR)

    def send_chunk(k, d):
        peer = lax.rem(my + d, N_DEV)
        sidx = (d - 1) * 2 + k

        @pl.when(my < 2)
        def _():
            pltpu.make_async_remote_copy(
                src_ref=buf.at[my, pl.ds(k * PPB, PPB)],
                dst_ref=buf.at[my, pl.ds(k * PPB, PPB)],
                send_sem=send_sems.at[sidx], recv_sem=recv_sems.at[my, k],
                device_id=(peer,), device_id_type=pl.DeviceIdType.MESH,
            ).start()

        @pl.when(my >= 2)
        def _():
            pltpu.make_async_remote_copy(
                src_ref=buf.at[my, pl.ds(k * PPB, PPB), pl.ds(0, GR), :],
                dst_ref=buf.at[my, pl.ds(k * PPB, PPB), pl.ds(0, GR), :],
                send_sem=send_sems.at[sidx], recv_sem=recv_sems.at[my, k],
                device_id=(peer,), device_id_type=pl.DeviceIdType.MESH,
            ).start()

    @pl.when(my == 0)
    def _():
        buf[pl.ds(my, 1), pl.ds(0, PPB)] = (
            _batch_planes(q, k_ref, v_ref, 0, SQ, None)[None])

    @pl.when(my == 1)
    def _():
        buf[pl.ds(my, 1), pl.ds(0, PPB)] = (
            _batch_planes(q, k_ref, v_ref, 0, SQ, mask1)[None])

    @pl.when(my >= 2)
    def _():
        buf[pl.ds(my, 1), pl.ds(0, PPB), pl.ds(0, GR), :] = (
            _batch_planes(q, k_ref, v_ref, 0, GR, None)[None])

    pl.semaphore_wait(barrier, N_DEV - 1)
    for d in d_order:
        send_chunk(0, d)

    @pl.when(my == 0)
    def _():
        buf[pl.ds(my, 1), pl.ds(PPB, PPB)] = (
            _batch_planes(q, k_ref, v_ref, 1, SQ, None)[None])

    @pl.when(my == 1)
    def _():
        buf[pl.ds(my, 1), pl.ds(PPB, PPB)] = (
            _batch_planes(q, k_ref, v_ref, 1, SQ, mask1)[None])

    @pl.when(my >= 2)
    def _():
        buf[pl.ds(my, 1), pl.ds(PPB, PPB), pl.ds(0, GR), :] = (
            _batch_planes(q, k_ref, v_ref, 1, GR, None)[None])

    for d in d_order:
        send_chunk(1, d)

    def read_stats(c_idx, b):
        st = buf[c_idx, b * PPB + HQ, :, :2 * HQ].astype(jnp.float32)
        return st[:, :HQ], st[:, HQ:]

    state = []
    for b in range(B):
        m_own, l_own = read_stats(my, b)
        ctx_own = [buf[my, b * PPB + h].astype(jnp.float32)
                   for h in range(HQ)]
        state.append({
            "mT": m_own[:GR], "denT": l_own[:GR],
            "numT": [c[:GR] for c in ctx_own],
            "mB": m_own[GR:], "denB": l_own[GR:],
            "numB": [c[GR:] for c in ctx_own],
        })
    neg = jnp.full((SQ - GR, HQ), -1e9, jnp.float32)
    zms = jnp.zeros((SQ - GR, HQ), jnp.float32)
    zn = jnp.zeros((SQ - GR, DH), jnp.float32)
    own_valid = my < 2
    for b in range(B):
        st = state[b]
        st["mB"] = jnp.where(own_valid, st["mB"], neg)
        st["denB"] = jnp.where(own_valid, st["denB"], zms)
        st["numB"] = [jnp.where(own_valid, n, zn) for n in st["numB"]]

    def incorporate(st, m_c, l_c, ctx_c):
        m_new = jnp.maximum(st["m"], m_c)
        a = jnp.exp(st["m"] - m_new)
        bb = jnp.exp(m_c - m_new)
        den = st["den"] * a + l_c * bb
        num = [st["num"][h] * a[:, h:h + 1] + ctx_c[h] * bb[:, h:h + 1]
               for h in range(HQ)]
        return {"m": m_new, "den": den, "num": num}

    for d in w_order:
        origin = lax.rem(my + d, N_DEV)
        for k in range(2):
            @pl.when(origin < 2)
            def _():
                pltpu.make_async_remote_copy(
                    src_ref=buf.at[origin, pl.ds(k * PPB, PPB)],
                    dst_ref=buf.at[origin, pl.ds(k * PPB, PPB)],
                    send_sem=send_sems.at[(d - 1) * 2 + k],
                    recv_sem=recv_sems.at[origin, k],
                    device_id=(origin,), device_id_type=pl.DeviceIdType.MESH,
                ).wait_recv()

            @pl.when(origin >= 2)
            def _():
                pltpu.make_async_remote_copy(
                    src_ref=buf.at[origin, pl.ds(k * PPB, PPB),
                                   pl.ds(0, GR), :],
                    dst_ref=buf.at[origin, pl.ds(k * PPB, PPB),
                                   pl.ds(0, GR), :],
                    send_sem=send_sems.at[(d - 1) * 2 + k],
                    recv_sem=recv_sems.at[origin, k],
                    device_id=(origin,), device_id_type=pl.DeviceIdType.MESH,
                ).wait_recv()

        is_full = origin < 2
        for b in range(B):
            m_c, l_c = read_stats(origin, b)
            ctx_c = [buf[origin, b * PPB + h].astype(jnp.float32)
                     for h in range(HQ)]
            st = state[b]
            top = incorporate(
                {"m": st["mT"], "den": st["denT"], "num": st["numT"]},
                m_c[:GR], l_c[:GR], [c[:GR] for c in ctx_c])
            st["mT"], st["denT"], st["numT"] = (
                top["m"], top["den"], top["num"])
            m_cb = jnp.where(is_full, m_c[GR:], neg)
            l_cb = jnp.where(is_full, l_c[GR:], zms)
            ctx_cb = [jnp.where(is_full, c[GR:], zn) for c in ctx_c]
            bot = incorporate(
                {"m": st["mB"], "den": st["denB"], "num": st["numB"]},
                m_cb, l_cb, ctx_cb)
            st["mB"], st["denB"], st["numB"] = (
                bot["m"], bot["den"], bot["num"])

    ctx_rows = []
    for b in range(B):
        st = state[b]
        heads = [jnp.concatenate(
            [st["numT"][h] / st["denT"][:, h:h + 1],
             st["numB"][h] / st["denB"][:, h:h + 1]], axis=0)
            for h in range(HQ)]
        ctx_rows.append(jnp.concatenate(heads, axis=1))
    ctx_all = jnp.concatenate(ctx_rows, axis=0)
    out_ref[...] = jnp.dot(ctx_all, wo_ref[...],
                           preferred_element_type=jnp.float32)

    for d in range(1, N_DEV):
        for k in range(2):
            @pl.when(my < 2)
            def _():
                pltpu.make_async_remote_copy(
                    src_ref=buf.at[my, pl.ds(k * PPB, PPB)],
                    dst_ref=buf.at[my, pl.ds(k * PPB, PPB)],
                    send_sem=send_sems.at[(d - 1) * 2 + k],
                    recv_sem=recv_sems.at[my, k],
                    device_id=(my,), device_id_type=pl.DeviceIdType.MESH,
                ).wait_send()

            @pl.when(my >= 2)
            def _():
                pltpu.make_async_remote_copy(
                    src_ref=buf.at[my, pl.ds(k * PPB, PPB),
                                   pl.ds(0, GR), :],
                    dst_ref=buf.at[my, pl.ds(k * PPB, PPB),
                                   pl.ds(0, GR), :],
                    send_sem=send_sems.at[(d - 1) * 2 + k],
                    recv_sem=recv_sems.at[my, k],
                    device_id=(my,), device_id_type=pl.DeviceIdType.MESH,
                ).wait_send()


def kernel(x, Wq, K_ext, V_ext, Wo):
    x2 = x.reshape(B * SQ, D_MODEL)
    k3 = K_ext.transpose(0, 2, 1, 3).reshape(B * HQ, SKV_SH, DH)
    v3 = V_ext.transpose(0, 2, 1, 3).reshape(B * HQ, SKV_SH, DH)

    out2 = pl.pallas_call(
        _body,
        out_shape=jax.ShapeDtypeStruct((B * SQ, D_MODEL), jnp.float32),
        in_specs=[pl.BlockSpec(memory_space=pltpu.VMEM)] * 5,
        out_specs=pl.BlockSpec(memory_space=pltpu.VMEM),
        scratch_shapes=[
            pltpu.VMEM((N_DEV, NP, SQ, DH), jnp.bfloat16),
            pltpu.SemaphoreType.DMA((2 * (N_DEV - 1),)),
            pltpu.SemaphoreType.DMA((N_DEV, 2)),
        ],
        compiler_params=pltpu.CompilerParams(collective_id=0),
    )(x2, Wq, k3, v3, Wo)
    return out2.reshape(B, SQ, D_MODEL)


# device time: 14845 ns/iter; 1.0600x vs baseline; 1.0600x over previous
import jax
import jax.numpy as jnp
from jax import lax
from jax.experimental import pallas as pl
from jax.experimental.pallas import tpu as pltpu

N_DEV = 4
B, SQ, SKV, HQ, DH = 2, 128, 512, 4, 64
D_MODEL = 512
SKV_SH = SKV // N_DEV
GR = 32
PPB = HQ + 1
NP = B * PPB


def _batch_planes(q, k_ref, v_ref, b, rows, mask):
    ctx_list, m_list, l_list = [], [], []
    for h in range(HQ):
        bh = b * HQ + h
        qbh = q[b * SQ:b * SQ + rows, h * DH:(h + 1) * DH]
        s = lax.dot_general(
            qbh, k_ref[bh], (((1,), (1,)), ((), ())),
            preferred_element_type=jnp.float32) * 0.125
        if mask is not None:
            s = jnp.where(mask, s, -1e9)
        m = jnp.max(s, axis=1, keepdims=True)
        w = jnp.exp(s - m)
        l = jnp.sum(w, axis=1, keepdims=True)
        ctx_list.append(jnp.dot(w, v_ref[bh],
                                preferred_element_type=jnp.float32))
        m_list.append(m)
        l_list.append(l)
    stats = jnp.concatenate(
        m_list + l_list + [jnp.zeros((rows, DH - 2 * HQ), jnp.float32)],
        axis=1)
    payload = jnp.concatenate([jnp.stack(ctx_list, axis=0), stats[None]],
                              axis=0)
    return payload.astype(jnp.bfloat16)


def _body(x_ref, wq_ref, k_ref, v_ref, wo_ref, out_ref,
          buf, send_sems, recv_sems):
    my = lax.axis_index("i")
    d_order = (2, 1, 3)

    barrier = pltpu.get_barrier_semaphore()
    for d in range(1, N_DEV):
        peer = lax.rem(my + d, N_DEV)
        pl.semaphore_signal(barrier, inc=1, device_id=(peer,),
                            device_id_type=pl.DeviceIdType.MESH)

    q = jnp.dot(x_ref[...], wq_ref[...],
                preferred_element_type=jnp.float32)

    qi = lax.broadcasted_iota(jnp.int32, (SQ, SKV_SH), 0)
    kj = lax.broadcasted_iota(jnp.int32, (SQ, SKV_SH), 1)
    mask1 = (kj <= qi) | (qi < GR)

    def send_chunk(k, d):
        peer = lax.rem(my + d, N_DEV)
        sidx = (d - 1) * 2 + k

        @pl.when(my < 2)
        def _():
            pltpu.make_async_remote_copy(
                src_ref=buf.at[my, pl.ds(k * PPB, PPB)],
                dst_ref=buf.at[my, pl.ds(k * PPB, PPB)],
                send_sem=send_sems.at[sidx], recv_sem=recv_sems.at[my, k],
                device_id=(peer,), device_id_type=pl.DeviceIdType.MESH,
            ).start()

        @pl.when(my >= 2)
        def _():
            pltpu.make_async_remote_copy(
                src_ref=buf.at[my, pl.ds(k * PPB, PPB), pl.ds(0, GR), :],
                dst_ref=buf.at[my, pl.ds(k * PPB, PPB), pl.ds(0, GR), :],
                send_sem=send_sems.at[sidx], recv_sem=recv_sems.at[my, k],
                device_id=(peer,), device_id_type=pl.DeviceIdType.MESH,
            ).start()

    @pl.when(my == 0)
    def _():
        buf[pl.ds(my, 1), pl.ds(0, PPB)] = (
            _batch_planes(q, k_ref, v_ref, 0, SQ, None)[None])

    @pl.when(my == 1)
    def _():
        buf[pl.ds(my, 1), pl.ds(0, PPB)] = (
            _batch_planes(q, k_ref, v_ref, 0, SQ, mask1)[None])

    @pl.when(my >= 2)
    def _():
        buf[pl.ds(my, 1), pl.ds(0, PPB), pl.ds(0, GR), :] = (
            _batch_planes(q, k_ref, v_ref, 0, GR, None)[None])

    pl.semaphore_wait(barrier, N_DEV - 1)
    for d in d_order:
        send_chunk(0, d)

    @pl.when(my == 0)
    def _():
        buf[pl.ds(my, 1), pl.ds(PPB, PPB)] = (
            _batch_planes(q, k_ref, v_ref, 1, SQ, None)[None])

    @pl.when(my == 1)
    def _():
        buf[pl.ds(my, 1), pl.ds(PPB, PPB)] = (
            _batch_planes(q, k_ref, v_ref, 1, SQ, mask1)[None])

    @pl.when(my >= 2)
    def _():
        buf[pl.ds(my, 1), pl.ds(PPB, PPB), pl.ds(0, GR), :] = (
            _batch_planes(q, k_ref, v_ref, 1, GR, None)[None])

    for d in d_order:
        send_chunk(1, d)

    for d in range(1, N_DEV):
        origin = lax.rem(my + d, N_DEV)
        for k in range(2):
            @pl.when(origin < 2)
            def _():
                pltpu.make_async_remote_copy(
                    src_ref=buf.at[origin, pl.ds(k * PPB, PPB)],
                    dst_ref=buf.at[origin, pl.ds(k * PPB, PPB)],
                    send_sem=send_sems.at[(d - 1) * 2 + k],
                    recv_sem=recv_sems.at[origin, k],
                    device_id=(origin,), device_id_type=pl.DeviceIdType.MESH,
                ).wait_recv()

            @pl.when(origin >= 2)
            def _():
                pltpu.make_async_remote_copy(
                    src_ref=buf.at[origin, pl.ds(k * PPB, PPB),
                                   pl.ds(0, GR), :],
                    dst_ref=buf.at[origin, pl.ds(k * PPB, PPB),
                                   pl.ds(0, GR), :],
                    send_sem=send_sems.at[(d - 1) * 2 + k],
                    recv_sem=recv_sems.at[origin, k],
                    device_id=(origin,), device_id_type=pl.DeviceIdType.MESH,
                ).wait_recv()

    ctx_rows = []
    for b in range(B):
        st = [buf[c, b * PPB + HQ, :, :2 * HQ].astype(jnp.float32)
              for c in range(N_DEV)]
        stA = [s[:GR] for s in st]
        stB = [st[c][GR:] for c in range(2)]
        mA = stA[0][:, :HQ]
        for c in range(1, N_DEV):
            mA = jnp.maximum(mA, stA[c][:, :HQ])
        sclA = [jnp.exp(stA[c][:, :HQ] - mA) for c in range(N_DEV)]
        denA = sum(stA[c][:, HQ:] * sclA[c] for c in range(N_DEV))
        mB = jnp.maximum(stB[0][:, :HQ], stB[1][:, :HQ])
        sclB = [jnp.exp(stB[c][:, :HQ] - mB) for c in range(2)]
        denB = sum(stB[c][:, HQ:] * sclB[c] for c in range(2))

        heads = []
        for h in range(HQ):
            p = b * PPB + h
            numA = sum(buf[c, p, :GR].astype(jnp.float32)
                       * sclA[c][:, h:h + 1]
                       for c in range(N_DEV))
            numB = sum(buf[c, p, GR:].astype(jnp.float32)
                       * sclB[c][:, h:h + 1]
                       for c in range(2))
            heads.append(jnp.concatenate(
                [numA / denA[:, h:h + 1], numB / denB[:, h:h + 1]],
                axis=0))
        ctx_rows.append(jnp.concatenate(heads, axis=1))
    ctx_all = jnp.concatenate(ctx_rows, axis=0)
    out_ref[...] = jnp.dot(ctx_all, wo_ref[...],
                           preferred_element_type=jnp.float32)

    for d in range(1, N_DEV):
        for k in range(2):
            @pl.when(my < 2)
            def _():
                pltpu.make_async_remote_copy(
                    src_ref=buf.at[my, pl.ds(k * PPB, PPB)],
                    dst_ref=buf.at[my, pl.ds(k * PPB, PPB)],
                    send_sem=send_sems.at[(d - 1) * 2 + k],
                    recv_sem=recv_sems.at[my, k],
                    device_id=(my,), device_id_type=pl.DeviceIdType.MESH,
                ).wait_send()

            @pl.when(my >= 2)
            def _():
                pltpu.make_async_remote_copy(
                    src_ref=buf.at[my, pl.ds(k * PPB, PPB),
                                   pl.ds(0, GR), :],
                    dst_ref=buf.at[my, pl.ds(k * PPB, PPB),
                                   pl.ds(0, GR), :],
                    send_sem=send_sems.at[(d - 1) * 2 + k],
                    recv_sem=recv_sems.at[my, k],
                    device_id=(my,), device_id_type=pl.DeviceIdType.MESH,
                ).wait_send()


def kernel(x, Wq, K_ext, V_ext, Wo):
    x2 = x.reshape(B * SQ, D_MODEL)
    k3 = K_ext.transpose(0, 2, 1, 3).reshape(B * HQ, SKV_SH, DH)
    v3 = V_ext.transpose(0, 2, 1, 3).reshape(B * HQ, SKV_SH, DH)

    out2 = pl.pallas_call(
        _body,
        out_shape=jax.ShapeDtypeStruct((B * SQ, D_MODEL), jnp.float32),
        in_specs=[pl.BlockSpec(memory_space=pltpu.VMEM)] * 5,
        out_specs=pl.BlockSpec(memory_space=pltpu.VMEM),
        scratch_shapes=[
            pltpu.VMEM((N_DEV, NP, SQ, DH), jnp.bfloat16),
            pltpu.SemaphoreType.DMA((2 * (N_DEV - 1),)),
            pltpu.SemaphoreType.DMA((N_DEV, 2)),
        ],
        compiler_params=pltpu.CompilerParams(collective_id=0),
    )(x2, Wq, k3, v3, Wo)
    return out2.reshape(B, SQ, D_MODEL)
